# Initial kernel scaffold; baseline (speedup 1.0000x reference)
import jax
import jax.numpy as jnp
from jax.experimental import pallas as pl


def kernel(node_features, edge_index, edge_attr_tensor, node_attr_scalar_raw, W_tp_msg, W_lin_msg, W_tp_upd, W_lin_upd):
    raise NotImplementedError("write your pallas kernel here")


# speedup vs baseline: 1.7245x; 1.7245x over previous
import functools

import jax
import jax.numpy as jnp
import numpy as np
from jax import lax
from jax.experimental import pallas as pl
from jax.experimental.pallas import tpu as pltpu
from jax.experimental.pallas import tpu_sc as plsc

N = 10000
E = 160000
D = 128
D_EDGE = 4
D_HID = 128

NC = 2
NS = 16
NW = NC * NS
EPW = E // NW
CHUNK = 40
NCHUNK = EPW // CHUNK
NPAD = 10240
ROWS_PER_SUB = NPAD // NS


def _gather_body(x_hbm, col_hbm, out_hbm, idx_v, buf, gsem):
    c = lax.axis_index("c")
    s = lax.axis_index("s")
    wid = s * NC + c
    pltpu.sync_copy(col_hbm.at[wid], idx_v)

    def step(i, _):
        pltpu.async_copy(x_hbm.at[idx_v.at[i]], buf, gsem).wait()
        pltpu.sync_copy(buf, out_hbm.at[pl.ds(wid * EPW + i * CHUNK, CHUNK), :])
        return 0

    lax.fori_loop(0, NCHUNK, step, 0)


@functools.lru_cache(maxsize=None)
def _gather_call():
    return functools.partial(
        pl.kernel,
        out_type=jax.ShapeDtypeStruct((E, D), jnp.float32),
        mesh=plsc.VectorSubcoreMesh(
            core_axis_name="c", subcore_axis_name="s", num_cores=NC, num_subcores=NS
        ),
        scratch_types=[
            pltpu.VMEM((NCHUNK, CHUNK), jnp.int32),
            pltpu.VMEM((CHUNK, D), jnp.float32),
            pltpu.SemaphoreType.DMA,
        ],
    )(_gather_body)


def _scatter_body(msg_hbm, row_hbm, zero_hbm, out_hbm, acc_sh, idx_v, buf, gsem):
    c = lax.axis_index("c")
    s = lax.axis_index("s")
    wid = s * NC + c
    pltpu.sync_copy(zero_hbm, acc_sh.at[pl.ds(s * ROWS_PER_SUB, ROWS_PER_SUB), :])
    pltpu.sync_copy(row_hbm.at[wid], idx_v)
    plsc.subcore_barrier()

    def step(i, _):
        pltpu.async_copy(
            msg_hbm.at[pl.ds(wid * EPW + i * CHUNK, CHUNK), :], buf, gsem
        ).wait()
        pltpu.sync_copy(buf, acc_sh.at[idx_v.at[i]], add=True)
        return 0

    lax.fori_loop(0, NCHUNK, step, 0)
    plsc.subcore_barrier()
    pltpu.sync_copy(
        acc_sh.at[pl.ds(s * ROWS_PER_SUB, ROWS_PER_SUB), :],
        out_hbm.at[c, pl.ds(s * ROWS_PER_SUB, ROWS_PER_SUB), :],
    )


@functools.lru_cache(maxsize=None)
def _scatter_call():
    return functools.partial(
        pl.kernel,
        out_type=jax.ShapeDtypeStruct((NC, NPAD, D), jnp.float32),
        mesh=plsc.VectorSubcoreMesh(
            core_axis_name="c", subcore_axis_name="s", num_cores=NC, num_subcores=NS
        ),
        scratch_types=[
            pltpu.VMEM_SHARED((NPAD, D), jnp.float32),
            pltpu.VMEM((NCHUNK, CHUNK), jnp.int32),
            pltpu.VMEM((CHUNK, D), jnp.float32),
            pltpu.SemaphoreType.DMA,
        ],
    )(_scatter_body)


BE = 2000


def _msg_body(xg_ref, ea_ref, w_ref, o_ref):
    x = xg_ref[...]
    acc = jnp.dot(x, w_ref[0], preferred_element_type=jnp.float32) * ea_ref[:, 0:1]
    for j in range(1, D_EDGE):
        acc += jnp.dot(x, w_ref[j], preferred_element_type=jnp.float32) * ea_ref[:, j : j + 1]
    y = acc * (1.0 / np.sqrt(D * D_EDGE))
    o_ref[...] = y * jax.nn.sigmoid(y)


def _msg_call(x_g, ea, w_t):
    return pl.pallas_call(
        _msg_body,
        grid=(E // BE,),
        in_specs=[
            pl.BlockSpec((BE, D), lambda i: (i, 0)),
            pl.BlockSpec((BE, D_EDGE), lambda i: (i, 0)),
            pl.BlockSpec((D_EDGE, D, D_HID), lambda i: (0, 0, 0)),
        ],
        out_specs=pl.BlockSpec((BE, D_HID), lambda i: (i, 0)),
        out_shape=jax.ShapeDtypeStruct((E, D_HID), jnp.float32),
    )(x_g, ea, w_t)


BN = 400
IC = 16
NIC = D // IC


def _upd_body(sp_ref, x_ref, wlm_ref, w2m_ref, wlu_ref, o_ref):
    s = sp_ref[0] + sp_ref[1]
    agg = jnp.dot(s, wlm_ref[...], preferred_element_type=jnp.float32) * (
        1.0 / np.sqrt(D_HID)
    )
    x = x_ref[...]
    acc = jnp.zeros((BN, D_HID), jnp.float32)
    for ic in range(NIC):
        t = jnp.dot(
            agg,
            w2m_ref[:, ic * IC * D_HID : (ic + 1) * IC * D_HID],
            preferred_element_type=jnp.float32,
        )
        t3 = t.reshape(BN, IC, D_HID)
        acc += jnp.sum(t3 * x[:, ic * IC : (ic + 1) * IC, None], axis=1)
    y = acc * (1.0 / np.sqrt(D * D_HID))
    y = y * jax.nn.sigmoid(y)
    o_ref[...] = x + jnp.dot(y, wlu_ref[...], preferred_element_type=jnp.float32) * (
        1.0 / np.sqrt(D_HID)
    )


def _upd_call(s_parts, x, w_lm, w2m, w_lu):
    return pl.pallas_call(
        _upd_body,
        grid=(N // BN,),
        in_specs=[
            pl.BlockSpec((NC, BN, D), lambda i: (0, i, 0)),
            pl.BlockSpec((BN, D), lambda i: (i, 0)),
            pl.BlockSpec((D_HID, D_HID), lambda i: (0, 0)),
            pl.BlockSpec((D_HID, D * D_HID), lambda i: (0, 0)),
            pl.BlockSpec((D_HID, D), lambda i: (0, 0)),
        ],
        out_specs=pl.BlockSpec((BN, D), lambda i: (i, 0)),
        out_shape=jax.ShapeDtypeStruct((N, D), jnp.float32),
    )(s_parts, x, w_lm, w2m, w_lu)


@jax.jit
def kernel(node_features, edge_index, edge_attr_tensor, node_attr_scalar_raw,
           W_tp_msg, W_lin_msg, W_tp_upd, W_lin_upd):
    del node_attr_scalar_raw
    row = edge_index[0].reshape(NW, NCHUNK, CHUNK)
    col = edge_index[1].reshape(NW, NCHUNK, CHUNK)
    w_msg_t = jnp.transpose(W_tp_msg, (1, 0, 2))
    w2m = jnp.transpose(W_tp_upd, (1, 0, 2)).reshape(D_HID, D * D_HID)
    zeros = jnp.zeros((ROWS_PER_SUB, D), jnp.float32)

    x_g = _gather_call()(node_features, col)
    msg = _msg_call(x_g, edge_attr_tensor, w_msg_t)
    s_parts = _scatter_call()(msg, row, zeros)
    return _upd_call(s_parts, node_features, W_lin_msg, w2m, W_lin_upd)


# speedup vs baseline: 2.2043x; 1.2782x over previous
import functools

import jax
import jax.numpy as jnp
import numpy as np
from jax import lax
from jax.experimental import pallas as pl
from jax.experimental.pallas import tpu as pltpu
from jax.experimental.pallas import tpu_sc as plsc

N = 10000
E = 160000
D = 128
D_EDGE = 4
D_HID = 128

NC = 2
NS = 16
NW = NC * NS
EPW = E // NW
CHUNK = 40
NCHUNK = EPW // CHUNK
ROWS_PER_SUB = 624


RING = 5
NROUND = NCHUNK // RING


def _gather_body(x_hbm, col_hbm, out_hbm, idx_v, bufs, gsem, wsem):
    c = lax.axis_index("c")
    s = lax.axis_index("s")
    wid = s * NC + c
    base = wid * EPW
    pltpu.sync_copy(col_hbm.at[wid], idx_v)

    def g_copy(i, slot):
        return pltpu.make_async_copy(x_hbm.at[idx_v.at[i]], bufs.at[slot], gsem.at[slot])

    def w_copy(i, slot):
        return pltpu.make_async_copy(
            bufs.at[slot], out_hbm.at[pl.ds(base + i * CHUNK, CHUNK), :], wsem.at[slot]
        )

    for r in range(RING):
        g_copy(r, r).start()

    def dround(o2, _):
        for half in range(2):
            o = o2 * 2 + half
            sb = half * RING
            sb2 = (1 - half) * RING
            for r in range(RING):
                i = o * RING + r
                g_copy(i, sb + r).wait()
                w_copy(i, sb + r).start()

                @pl.when(o >= 1)
                def _(i=i, nxt=sb2 + r):
                    w_copy(i - RING, nxt).wait()

                g_copy(i + RING, sb2 + r).start()
        return 0

    lax.fori_loop(0, (NROUND - 1) // 2, dround, 0)
    for r in range(RING):
        i = (NROUND - 1) * RING + r
        g_copy(i, r).wait()
        w_copy(i, r).start()
        w_copy(i, r).wait()
        w_copy(i - RING, RING + r).wait()


@functools.lru_cache(maxsize=None)
def _gather_call():
    return functools.partial(
        pl.kernel,
        out_type=jax.ShapeDtypeStruct((E, D), jnp.float32),
        mesh=plsc.VectorSubcoreMesh(
            core_axis_name="c", subcore_axis_name="s", num_cores=NC, num_subcores=NS
        ),
        scratch_types=[
            pltpu.VMEM((NCHUNK, CHUNK), jnp.int32),
            pltpu.VMEM((2 * RING, CHUNK, D), jnp.float32),
            pltpu.SemaphoreType.DMA((2 * RING,)),
            pltpu.SemaphoreType.DMA((2 * RING,)),
        ],
    )(_gather_body)


def _scatter_body(msg_hbm, row_hbm, zero_hbm, out_hbm, acc_sh, idx_v, bufs, lsem, ssem):
    c = lax.axis_index("c")
    s = lax.axis_index("s")
    wid = s * NC + c
    base = wid * EPW
    pltpu.sync_copy(zero_hbm, acc_sh.at[pl.ds(s * ROWS_PER_SUB, ROWS_PER_SUB), :])

    @pl.when(s == NS - 1)
    def _():
        pltpu.sync_copy(
            zero_hbm.at[pl.ds(0, 16), :], acc_sh.at[pl.ds(NS * ROWS_PER_SUB, 16), :]
        )

    pltpu.sync_copy(row_hbm.at[wid], idx_v)
    plsc.subcore_barrier()

    def l_copy(i, slot):
        return pltpu.make_async_copy(
            msg_hbm.at[pl.ds(base + i * CHUNK, CHUNK), :], bufs.at[slot], lsem.at[slot]
        )

    def s_copy(i, slot):
        return pltpu.make_async_copy(
            bufs.at[slot], acc_sh.at[idx_v.at[i]], ssem.at[slot]
        )

    for r in range(RING):
        l_copy(r, r).start()

    def rnd(o, _):
        for r in range(RING):
            i = o * RING + r
            l_copy(i, r).wait()
            pltpu.async_copy(bufs.at[r], acc_sh.at[idx_v.at[i]], ssem.at[r], add=True)
            s_copy(i, r).wait()
            l_copy(i + RING, r).start()
        return 0

    lax.fori_loop(0, NROUND - 1, rnd, 0)
    for r in range(RING):
        i = (NROUND - 1) * RING + r
        l_copy(i, r).wait()
        pltpu.async_copy(bufs.at[r], acc_sh.at[idx_v.at[i]], ssem.at[r], add=True)
        s_copy(i, r).wait()
    plsc.subcore_barrier()
    pltpu.sync_copy(
        acc_sh.at[pl.ds(s * ROWS_PER_SUB, ROWS_PER_SUB), :],
        out_hbm.at[c, pl.ds(s * ROWS_PER_SUB, ROWS_PER_SUB), :],
    )

    @pl.when(s == NS - 1)
    def _():
        pltpu.sync_copy(
            acc_sh.at[pl.ds(NS * ROWS_PER_SUB, 16), :],
            out_hbm.at[c, pl.ds(NS * ROWS_PER_SUB, 16), :],
        )


@functools.lru_cache(maxsize=None)
def _scatter_call():
    return functools.partial(
        pl.kernel,
        out_type=jax.ShapeDtypeStruct((NC, N, D), jnp.float32),
        mesh=plsc.VectorSubcoreMesh(
            core_axis_name="c", subcore_axis_name="s", num_cores=NC, num_subcores=NS
        ),
        scratch_types=[
            pltpu.VMEM_SHARED((N, D), jnp.float32),
            pltpu.VMEM((NCHUNK, CHUNK), jnp.int32),
            pltpu.VMEM((RING, CHUNK, D), jnp.float32),
            pltpu.SemaphoreType.DMA((RING,)),
            pltpu.SemaphoreType.DMA((RING,)),
        ],
    )(_scatter_body)


BE = 2000


def _msg_body(xg_ref, ea_ref, w_ref, o_ref):
    x = xg_ref[...]
    acc = jnp.dot(x, w_ref[0], preferred_element_type=jnp.float32) * ea_ref[:, 0:1]
    for j in range(1, D_EDGE):
        acc += jnp.dot(x, w_ref[j], preferred_element_type=jnp.float32) * ea_ref[:, j : j + 1]
    y = acc * (1.0 / np.sqrt(D * D_EDGE))
    o_ref[...] = y * jax.nn.sigmoid(y)


def _msg_call(x_g, ea, w_t):
    return pl.pallas_call(
        _msg_body,
        grid=(E // BE,),
        in_specs=[
            pl.BlockSpec((BE, D), lambda i: (i, 0)),
            pl.BlockSpec((BE, D_EDGE), lambda i: (i, 0)),
            pl.BlockSpec((D_EDGE, D, D_HID), lambda i: (0, 0, 0)),
        ],
        out_specs=pl.BlockSpec((BE, D_HID), lambda i: (i, 0)),
        out_shape=jax.ShapeDtypeStruct((E, D_HID), jnp.float32),
    )(x_g, ea, w_t)


BN = 400
IC = 16
NIC = D // IC


def _upd_body(sp_ref, x_ref, wlm_ref, w2m_ref, wlu_ref, o_ref):
    s = sp_ref[0] + sp_ref[1]
    agg = jnp.dot(s, wlm_ref[...], preferred_element_type=jnp.float32) * (
        1.0 / np.sqrt(D_HID)
    )
    x = x_ref[...]
    acc = jnp.zeros((BN, D_HID), jnp.float32)
    for ic in range(NIC):
        t = jnp.dot(
            agg,
            w2m_ref[:, ic * IC * D_HID : (ic + 1) * IC * D_HID],
            preferred_element_type=jnp.float32,
        )
        t3 = t.reshape(BN, IC, D_HID)
        acc += jnp.sum(t3 * x[:, ic * IC : (ic + 1) * IC, None], axis=1)
    y = acc * (1.0 / np.sqrt(D * D_HID))
    y = y * jax.nn.sigmoid(y)
    o_ref[...] = x + jnp.dot(y, wlu_ref[...], preferred_element_type=jnp.float32) * (
        1.0 / np.sqrt(D_HID)
    )


def _upd_call(s_parts, x, w_lm, w2m, w_lu):
    return pl.pallas_call(
        _upd_body,
        grid=(N // BN,),
        in_specs=[
            pl.BlockSpec((NC, BN, D), lambda i: (0, i, 0)),
            pl.BlockSpec((BN, D), lambda i: (i, 0)),
            pl.BlockSpec((D_HID, D_HID), lambda i: (0, 0)),
            pl.BlockSpec((D_HID, D * D_HID), lambda i: (0, 0)),
            pl.BlockSpec((D_HID, D), lambda i: (0, 0)),
        ],
        out_specs=pl.BlockSpec((BN, D), lambda i: (i, 0)),
        out_shape=jax.ShapeDtypeStruct((N, D), jnp.float32),
    )(s_parts, x, w_lm, w2m, w_lu)


@jax.jit
def kernel(node_features, edge_index, edge_attr_tensor, node_attr_scalar_raw,
           W_tp_msg, W_lin_msg, W_tp_upd, W_lin_upd):
    del node_attr_scalar_raw
    row = edge_index[0].reshape(NW, NCHUNK, CHUNK)
    col = edge_index[1].reshape(NW, NCHUNK, CHUNK)
    w_msg_t = jnp.transpose(W_tp_msg, (1, 0, 2))
    w2m = jnp.transpose(W_tp_upd, (1, 0, 2)).reshape(D_HID, D * D_HID)
    zeros = jnp.zeros((ROWS_PER_SUB, D), jnp.float32)

    x_g = _gather_call()(node_features, col)
    msg = _msg_call(x_g, edge_attr_tensor, w_msg_t)
    s_parts = _scatter_call()(msg, row, zeros)
    return _upd_call(s_parts, node_features, W_lin_msg, w2m, W_lin_upd)


# speedup vs baseline: 2.3277x; 1.0560x over previous
import functools

import jax
import jax.numpy as jnp
import numpy as np
from jax import lax
from jax.experimental import pallas as pl
from jax.experimental.pallas import tpu as pltpu
from jax.experimental.pallas import tpu_sc as plsc

N = 10000
E = 160000
D = 128
D_EDGE = 4
D_HID = 128

NC = 2
NS = 16
NW = NC * NS
EPW = E // NW
CHUNK = 40
NCHUNK = EPW // CHUNK
ROWS_PER_SUB = 624


RING = 5
NROUND = NCHUNK // RING


def _gather_body(x_hbm, col_hbm, out_hbm, idx_v, bufs, gsem, wsem):
    c = lax.axis_index("c")
    s = lax.axis_index("s")
    wid = s * NC + c
    base = wid * EPW
    pltpu.sync_copy(col_hbm.at[wid], idx_v)

    def g_copy(i, slot):
        return pltpu.make_async_copy(x_hbm.at[idx_v.at[i]], bufs.at[slot], gsem.at[slot])

    def w_copy(i, slot):
        return pltpu.make_async_copy(
            bufs.at[slot], out_hbm.at[pl.ds(base + i * CHUNK, CHUNK), :], wsem.at[slot]
        )

    for r in range(RING):
        g_copy(r, r).start()

    def dround(o2, _):
        for half in range(2):
            o = o2 * 2 + half
            sb = half * RING
            sb2 = (1 - half) * RING
            for r in range(RING):
                i = o * RING + r
                g_copy(i, sb + r).wait()
                w_copy(i, sb + r).start()

                @pl.when(o >= 1)
                def _(i=i, nxt=sb2 + r):
                    w_copy(i - RING, nxt).wait()

                g_copy(i + RING, sb2 + r).start()
        return 0

    lax.fori_loop(0, (NROUND - 1) // 2, dround, 0)
    for r in range(RING):
        i = (NROUND - 1) * RING + r
        g_copy(i, r).wait()
        w_copy(i, r).start()
        w_copy(i, r).wait()
        w_copy(i - RING, RING + r).wait()


@functools.lru_cache(maxsize=None)
def _gather_call():
    return functools.partial(
        pl.kernel,
        out_type=jax.ShapeDtypeStruct((E, D), jnp.float32),
        mesh=plsc.VectorSubcoreMesh(
            core_axis_name="c", subcore_axis_name="s", num_cores=NC, num_subcores=NS
        ),
        scratch_types=[
            pltpu.VMEM((NCHUNK, CHUNK), jnp.int32),
            pltpu.VMEM((2 * RING, CHUNK, D), jnp.float32),
            pltpu.SemaphoreType.DMA((2 * RING,)),
            pltpu.SemaphoreType.DMA((2 * RING,)),
        ],
    )(_gather_body)


def _scatter_body(msg_hbm, row_hbm, zero_hbm, out_hbm, acc_sh, idx_v, bufs, lsem, ssem):
    c = lax.axis_index("c")
    s = lax.axis_index("s")
    wid = s * NC + c
    base = wid * EPW
    pltpu.sync_copy(zero_hbm, acc_sh.at[pl.ds(s * ROWS_PER_SUB, ROWS_PER_SUB), :])

    @pl.when(s == NS - 1)
    def _():
        pltpu.sync_copy(
            zero_hbm.at[pl.ds(0, 16), :], acc_sh.at[pl.ds(NS * ROWS_PER_SUB, 16), :]
        )

    pltpu.sync_copy(row_hbm.at[wid], idx_v)
    plsc.subcore_barrier()

    def l_copy(i, slot):
        return pltpu.make_async_copy(
            msg_hbm.at[pl.ds(base + i * CHUNK, CHUNK), :], bufs.at[slot], lsem.at[slot]
        )

    def s_copy(i, slot):
        return pltpu.make_async_copy(
            bufs.at[slot], acc_sh.at[idx_v.at[i]], ssem.at[slot]
        )

    for r in range(RING):
        l_copy(r, r).start()

    def rnd(o, _):
        for r in range(RING):
            i = o * RING + r
            l_copy(i, r).wait()
            pltpu.async_copy(bufs.at[r], acc_sh.at[idx_v.at[i]], ssem.at[r], add=True)
            s_copy(i, r).wait()
            l_copy(i + RING, r).start()
        return 0

    lax.fori_loop(0, NROUND - 1, rnd, 0)
    for r in range(RING):
        i = (NROUND - 1) * RING + r
        l_copy(i, r).wait()
        pltpu.async_copy(bufs.at[r], acc_sh.at[idx_v.at[i]], ssem.at[r], add=True)
        s_copy(i, r).wait()
    plsc.subcore_barrier()
    pltpu.sync_copy(
        acc_sh.at[pl.ds(s * ROWS_PER_SUB, ROWS_PER_SUB), :],
        out_hbm.at[c, pl.ds(s * ROWS_PER_SUB, ROWS_PER_SUB), :],
    )

    @pl.when(s == NS - 1)
    def _():
        pltpu.sync_copy(
            acc_sh.at[pl.ds(NS * ROWS_PER_SUB, 16), :],
            out_hbm.at[c, pl.ds(NS * ROWS_PER_SUB, 16), :],
        )


@functools.lru_cache(maxsize=None)
def _scatter_call():
    return functools.partial(
        pl.kernel,
        out_type=jax.ShapeDtypeStruct((NC, N, D), jnp.float32),
        mesh=plsc.VectorSubcoreMesh(
            core_axis_name="c", subcore_axis_name="s", num_cores=NC, num_subcores=NS
        ),
        scratch_types=[
            pltpu.VMEM_SHARED((N, D), jnp.float32),
            pltpu.VMEM((NCHUNK, CHUNK), jnp.int32),
            pltpu.VMEM((RING, CHUNK, D), jnp.float32),
            pltpu.SemaphoreType.DMA((RING,)),
            pltpu.SemaphoreType.DMA((RING,)),
        ],
    )(_scatter_body)


BE = 2000


def _msg_body(xg_ref, ea_ref, w_ref, o_ref):
    x = xg_ref[...]
    ym = (ea_ref[...][:, :, None] * x[:, None, :]).reshape(BE, D_EDGE * D)
    y = jnp.dot(ym, w_ref[...], preferred_element_type=jnp.float32) * (
        1.0 / np.sqrt(D * D_EDGE)
    )
    o_ref[...] = y * jax.nn.sigmoid(y)


def _msg_call(x_g, ea, w_t):
    return pl.pallas_call(
        _msg_body,
        grid=(E // BE,),
        in_specs=[
            pl.BlockSpec((BE, D), lambda i: (i, 0)),
            pl.BlockSpec((BE, D_EDGE), lambda i: (i, 0)),
            pl.BlockSpec((D_EDGE * D, D_HID), lambda i: (0, 0)),
        ],
        out_specs=pl.BlockSpec((BE, D_HID), lambda i: (i, 0)),
        out_shape=jax.ShapeDtypeStruct((E, D_HID), jnp.float32),
    )(x_g, ea, w_t)


BN = 400
IC = 16
NIC = D // IC


def _upd_body(sp_ref, x_ref, wlm_ref, w2f_ref, wlu_ref, o_ref):
    s = sp_ref[0] + sp_ref[1]
    agg = jnp.dot(s, wlm_ref[...], preferred_element_type=jnp.float32) * (
        1.0 / np.sqrt(D_HID)
    )
    x = x_ref[...]
    acc = jnp.zeros((BN, D_HID), jnp.float32)
    for ic in range(NIC):
        yc = (x[:, ic * IC : (ic + 1) * IC, None] * agg[:, None, :]).reshape(
            BN, IC * D_HID
        )
        acc += jnp.dot(yc, w2f_ref[ic], preferred_element_type=jnp.float32)
    y = acc * (1.0 / np.sqrt(D * D_HID))
    y = y * jax.nn.sigmoid(y)
    o_ref[...] = x + jnp.dot(y, wlu_ref[...], preferred_element_type=jnp.float32) * (
        1.0 / np.sqrt(D_HID)
    )


def _upd_call(s_parts, x, w_lm, w2f, w_lu):
    return pl.pallas_call(
        _upd_body,
        grid=(N // BN,),
        in_specs=[
            pl.BlockSpec((NC, BN, D), lambda i: (0, i, 0)),
            pl.BlockSpec((BN, D), lambda i: (i, 0)),
            pl.BlockSpec((D_HID, D_HID), lambda i: (0, 0)),
            pl.BlockSpec((NIC, IC * D_HID, D_HID), lambda i: (0, 0, 0)),
            pl.BlockSpec((D_HID, D), lambda i: (0, 0)),
        ],
        out_specs=pl.BlockSpec((BN, D), lambda i: (i, 0)),
        out_shape=jax.ShapeDtypeStruct((N, D), jnp.float32),
    )(s_parts, x, w_lm, w2f, w_lu)


@jax.jit
def kernel(node_features, edge_index, edge_attr_tensor, node_attr_scalar_raw,
           W_tp_msg, W_lin_msg, W_tp_upd, W_lin_upd):
    del node_attr_scalar_raw
    row = edge_index[0].reshape(NW, NCHUNK, CHUNK)
    col = edge_index[1].reshape(NW, NCHUNK, CHUNK)
    w_msg_t = jnp.transpose(W_tp_msg, (1, 0, 2)).reshape(D_EDGE * D, D_HID)
    w2f = W_tp_upd.reshape(NIC, IC * D_HID, D_HID)
    zeros = jnp.zeros((ROWS_PER_SUB, D), jnp.float32)

    x_g = _gather_call()(node_features, col)
    msg = _msg_call(x_g, edge_attr_tensor, w_msg_t)
    s_parts = _scatter_call()(msg, row, zeros)
    return _upd_call(s_parts, node_features, W_lin_msg, w2f, W_lin_upd)


# speedup vs baseline: 2.6367x; 1.1327x over previous
import functools

import jax
import jax.numpy as jnp
import numpy as np
from jax import lax
from jax.experimental import pallas as pl
from jax.experimental.pallas import tpu as pltpu
from jax.experimental.pallas import tpu_sc as plsc

N = 10000
E = 160000
D = 128
D_EDGE = 4
D_HID = 128

NC = 2
NS = 16
NW = NC * NS
EPW = E // NW
CHUNK = 40
NCHUNK = EPW // CHUNK
ROWS_PER_SUB = 624


RING = 5
NROUND = NCHUNK // RING


def _gather_body(x_hbm, col_hbm, out_hbm, idx_v, bufs, gsem, wsem):
    c = lax.axis_index("c")
    s = lax.axis_index("s")
    wid = s * NC + c
    base = wid * EPW
    pltpu.sync_copy(col_hbm.at[wid], idx_v)

    def g_copy(i, slot):
        return pltpu.make_async_copy(x_hbm.at[idx_v.at[i]], bufs.at[slot], gsem.at[slot])

    def w_copy(i, slot):
        return pltpu.make_async_copy(
            bufs.at[slot], out_hbm.at[pl.ds(base + i * CHUNK, CHUNK), :], wsem.at[slot]
        )

    for r in range(RING):
        g_copy(r, r).start()

    def dround(o2, _):
        for half in range(2):
            o = o2 * 2 + half
            sb = half * RING
            sb2 = (1 - half) * RING
            for r in range(RING):
                i = o * RING + r
                g_copy(i, sb + r).wait()
                w_copy(i, sb + r).start()

                @pl.when(o >= 1)
                def _(i=i, nxt=sb2 + r):
                    w_copy(i - RING, nxt).wait()

                g_copy(i + RING, sb2 + r).start()
        return 0

    lax.fori_loop(0, (NROUND - 1) // 2, dround, 0)
    for r in range(RING):
        i = (NROUND - 1) * RING + r
        g_copy(i, r).wait()
        w_copy(i, r).start()
        w_copy(i, r).wait()
        w_copy(i - RING, RING + r).wait()


@functools.lru_cache(maxsize=None)
def _gather_call():
    return functools.partial(
        pl.kernel,
        out_type=jax.ShapeDtypeStruct((E, D), jnp.float32),
        mesh=plsc.VectorSubcoreMesh(
            core_axis_name="c", subcore_axis_name="s", num_cores=NC, num_subcores=NS
        ),
        scratch_types=[
            pltpu.VMEM((NCHUNK, CHUNK), jnp.int32),
            pltpu.VMEM((2 * RING, CHUNK, D), jnp.float32),
            pltpu.SemaphoreType.DMA((2 * RING,)),
            pltpu.SemaphoreType.DMA((2 * RING,)),
        ],
    )(_gather_body)


def _scatter_body(msg_hbm, row_hbm, zero_hbm, out_hbm, acc_sh, idx_v, bufs, lsem, ssem):
    c = lax.axis_index("c")
    s = lax.axis_index("s")
    wid = s * NC + c
    base = wid * EPW
    pltpu.sync_copy(zero_hbm, acc_sh.at[pl.ds(s * ROWS_PER_SUB, ROWS_PER_SUB), :])

    @pl.when(s == NS - 1)
    def _():
        pltpu.sync_copy(
            zero_hbm.at[pl.ds(0, 16), :], acc_sh.at[pl.ds(NS * ROWS_PER_SUB, 16), :]
        )

    pltpu.sync_copy(row_hbm.at[wid], idx_v)
    plsc.subcore_barrier()

    def l_copy(i, slot):
        return pltpu.make_async_copy(
            msg_hbm.at[pl.ds(base + i * CHUNK, CHUNK), :], bufs.at[slot], lsem.at[slot]
        )

    def s_copy(i, slot):
        return pltpu.make_async_copy(
            bufs.at[slot], acc_sh.at[idx_v.at[i]], ssem.at[slot]
        )

    for r in range(RING):
        l_copy(r, r).start()

    def rnd(o, _):
        for r in range(RING):
            i = o * RING + r
            l_copy(i, r).wait()
            pltpu.async_copy(bufs.at[r], acc_sh.at[idx_v.at[i]], ssem.at[r], add=True)
            s_copy(i, r).wait()
            l_copy(i + RING, r).start()
        return 0

    lax.fori_loop(0, NROUND - 1, rnd, 0)
    for r in range(RING):
        i = (NROUND - 1) * RING + r
        l_copy(i, r).wait()
        pltpu.async_copy(bufs.at[r], acc_sh.at[idx_v.at[i]], ssem.at[r], add=True)
        s_copy(i, r).wait()
    plsc.subcore_barrier()
    pltpu.sync_copy(
        acc_sh.at[pl.ds(s * ROWS_PER_SUB, ROWS_PER_SUB), :],
        out_hbm.at[c, pl.ds(s * ROWS_PER_SUB, ROWS_PER_SUB), :],
    )

    @pl.when(s == NS - 1)
    def _():
        pltpu.sync_copy(
            acc_sh.at[pl.ds(NS * ROWS_PER_SUB, 16), :],
            out_hbm.at[c, pl.ds(NS * ROWS_PER_SUB, 16), :],
        )


@functools.lru_cache(maxsize=None)
def _scatter_call():
    return functools.partial(
        pl.kernel,
        out_type=jax.ShapeDtypeStruct((NC, N, D), jnp.float32),
        mesh=plsc.VectorSubcoreMesh(
            core_axis_name="c", subcore_axis_name="s", num_cores=NC, num_subcores=NS
        ),
        scratch_types=[
            pltpu.VMEM_SHARED((N, D), jnp.float32),
            pltpu.VMEM((NCHUNK, CHUNK), jnp.int32),
            pltpu.VMEM((RING, CHUNK, D), jnp.float32),
            pltpu.SemaphoreType.DMA((RING,)),
            pltpu.SemaphoreType.DMA((RING,)),
        ],
    )(_scatter_body)


BE = 2000


def _msg_body(xg_ref, ea_ref, w_ref, o_ref):
    x = xg_ref[...]
    eab = jnp.concatenate(
        [jnp.broadcast_to(ea_ref[:, j : j + 1], (BE, D)) for j in range(D_EDGE)],
        axis=1,
    )
    ym = eab * jnp.tile(x, (1, D_EDGE))
    y = jnp.dot(ym, w_ref[...], preferred_element_type=jnp.float32) * (
        1.0 / np.sqrt(D * D_EDGE)
    )
    o_ref[...] = y * jax.nn.sigmoid(y)


def _msg_call(x_g, ea, w_t):
    return pl.pallas_call(
        _msg_body,
        grid=(E // BE,),
        in_specs=[
            pl.BlockSpec((BE, D), lambda i: (i, 0)),
            pl.BlockSpec((BE, D_EDGE), lambda i: (i, 0)),
            pl.BlockSpec((D_EDGE * D, D_HID), lambda i: (0, 0)),
        ],
        out_specs=pl.BlockSpec((BE, D_HID), lambda i: (i, 0)),
        out_shape=jax.ShapeDtypeStruct((E, D_HID), jnp.float32),
    )(x_g, ea, w_t)


BN = 400
IC = 16
NIC = D // IC


def _upd_body(sp_ref, x_ref, wlm_ref, w2f_ref, wlu_ref, o_ref):
    s = sp_ref[0] + sp_ref[1]
    agg = jnp.dot(s, wlm_ref[...], preferred_element_type=jnp.float32) * (
        1.0 / np.sqrt(D_HID)
    )
    x = x_ref[...]
    acc = jnp.zeros((BN, D_HID), jnp.float32)
    agg_t = jnp.tile(agg, (1, IC))
    for ic in range(NIC):
        xb = jnp.concatenate(
            [
                jnp.broadcast_to(x[:, ic * IC + i : ic * IC + i + 1], (BN, D_HID))
                for i in range(IC)
            ],
            axis=1,
        )
        acc += jnp.dot(xb * agg_t, w2f_ref[ic], preferred_element_type=jnp.float32)
    y = acc * (1.0 / np.sqrt(D * D_HID))
    y = y * jax.nn.sigmoid(y)
    o_ref[...] = x + jnp.dot(y, wlu_ref[...], preferred_element_type=jnp.float32) * (
        1.0 / np.sqrt(D_HID)
    )


def _upd_call(s_parts, x, w_lm, w2f, w_lu):
    return pl.pallas_call(
        _upd_body,
        grid=(N // BN,),
        in_specs=[
            pl.BlockSpec((NC, BN, D), lambda i: (0, i, 0)),
            pl.BlockSpec((BN, D), lambda i: (i, 0)),
            pl.BlockSpec((D_HID, D_HID), lambda i: (0, 0)),
            pl.BlockSpec((NIC, IC * D_HID, D_HID), lambda i: (0, 0, 0)),
            pl.BlockSpec((D_HID, D), lambda i: (0, 0)),
        ],
        out_specs=pl.BlockSpec((BN, D), lambda i: (i, 0)),
        out_shape=jax.ShapeDtypeStruct((N, D), jnp.float32),
    )(s_parts, x, w_lm, w2f, w_lu)


@jax.jit
def kernel(node_features, edge_index, edge_attr_tensor, node_attr_scalar_raw,
           W_tp_msg, W_lin_msg, W_tp_upd, W_lin_upd):
    del node_attr_scalar_raw
    row = edge_index[0].reshape(NW, NCHUNK, CHUNK)
    col = edge_index[1].reshape(NW, NCHUNK, CHUNK)
    w_msg_t = jnp.transpose(W_tp_msg, (1, 0, 2)).reshape(D_EDGE * D, D_HID)
    w2f = W_tp_upd.reshape(NIC, IC * D_HID, D_HID)
    zeros = jnp.zeros((ROWS_PER_SUB, D), jnp.float32)

    x_g = _gather_call()(node_features, col)
    msg = _msg_call(x_g, edge_attr_tensor, w_msg_t)
    s_parts = _scatter_call()(msg, row, zeros)
    return _upd_call(s_parts, node_features, W_lin_msg, w2f, W_lin_upd)


# speedup vs baseline: 3.0354x; 1.1512x over previous
import functools

import jax
import jax.numpy as jnp
import numpy as np
from jax import lax
from jax.experimental import pallas as pl
from jax.experimental.pallas import tpu as pltpu
from jax.experimental.pallas import tpu_sc as plsc

N = 10000
E = 160000
D = 128
D_EDGE = 4
D_HID = 128

NC = 2
NS = 16
NW = NC * NS
EPW = E // NW
CHUNK = 40
NCHUNK = EPW // CHUNK
ROWS_PER_SUB = 624


RING = 5
NROUND = NCHUNK // RING


def _gather_body(x_hbm, col_hbm, out_hbm, idx_v, bufs, gsem, wsem):
    c = lax.axis_index("c")
    s = lax.axis_index("s")
    wid = s * NC + c
    base = wid * EPW
    pltpu.sync_copy(col_hbm.at[wid], idx_v)

    def g_copy(i, slot):
        return pltpu.make_async_copy(x_hbm.at[idx_v.at[i]], bufs.at[slot], gsem.at[slot])

    def w_copy(i, slot):
        return pltpu.make_async_copy(
            bufs.at[slot], out_hbm.at[pl.ds(base + i * CHUNK, CHUNK), :], wsem.at[slot]
        )

    for r in range(RING):
        g_copy(r, r).start()

    def dround(o2, _):
        for half in range(2):
            o = o2 * 2 + half
            sb = half * RING
            sb2 = (1 - half) * RING
            for r in range(RING):
                i = o * RING + r
                g_copy(i, sb + r).wait()
                w_copy(i, sb + r).start()

                @pl.when(o >= 1)
                def _(i=i, nxt=sb2 + r):
                    w_copy(i - RING, nxt).wait()

                g_copy(i + RING, sb2 + r).start()
        return 0

    lax.fori_loop(0, (NROUND - 1) // 2, dround, 0)
    for r in range(RING):
        i = (NROUND - 1) * RING + r
        g_copy(i, r).wait()
        w_copy(i, r).start()
        w_copy(i, r).wait()
        w_copy(i - RING, RING + r).wait()


@functools.lru_cache(maxsize=None)
def _gather_call():
    return functools.partial(
        pl.kernel,
        out_type=jax.ShapeDtypeStruct((E, D), jnp.float32),
        mesh=plsc.VectorSubcoreMesh(
            core_axis_name="c", subcore_axis_name="s", num_cores=NC, num_subcores=NS
        ),
        scratch_types=[
            pltpu.VMEM((NCHUNK, CHUNK), jnp.int32),
            pltpu.VMEM((2 * RING, CHUNK, D), jnp.float32),
            pltpu.SemaphoreType.DMA((2 * RING,)),
            pltpu.SemaphoreType.DMA((2 * RING,)),
        ],
    )(_gather_body)


def _scatter_body(msg_hbm, row_hbm, zero_hbm, out_hbm, acc_sh, idx_v, bufs, lsem, ssem):
    c = lax.axis_index("c")
    s = lax.axis_index("s")
    wid = s * NC + c
    base = wid * EPW
    pltpu.sync_copy(zero_hbm, acc_sh.at[pl.ds(s * ROWS_PER_SUB, ROWS_PER_SUB), :])

    @pl.when(s == NS - 1)
    def _():
        pltpu.sync_copy(
            zero_hbm.at[pl.ds(0, 16), :], acc_sh.at[pl.ds(NS * ROWS_PER_SUB, 16), :]
        )

    pltpu.sync_copy(row_hbm.at[wid], idx_v)
    plsc.subcore_barrier()

    def l_copy(i, slot):
        return pltpu.make_async_copy(
            msg_hbm.at[pl.ds(base + i * CHUNK, CHUNK), :], bufs.at[slot], lsem.at[slot]
        )

    def s_copy(i, slot):
        return pltpu.make_async_copy(
            bufs.at[slot], acc_sh.at[idx_v.at[i]], ssem.at[slot]
        )

    for r in range(RING):
        l_copy(r, r).start()

    def rnd(o, _):
        for r in range(RING):
            i = o * RING + r
            l_copy(i, r).wait()
            pltpu.async_copy(bufs.at[r], acc_sh.at[idx_v.at[i]], ssem.at[r], add=True)
            s_copy(i, r).wait()
            l_copy(i + RING, r).start()
        return 0

    lax.fori_loop(0, NROUND - 1, rnd, 0)
    for r in range(RING):
        i = (NROUND - 1) * RING + r
        l_copy(i, r).wait()
        pltpu.async_copy(bufs.at[r], acc_sh.at[idx_v.at[i]], ssem.at[r], add=True)
        s_copy(i, r).wait()
    plsc.subcore_barrier()
    pltpu.sync_copy(
        acc_sh.at[pl.ds(s * ROWS_PER_SUB, ROWS_PER_SUB), :],
        out_hbm.at[c, pl.ds(s * ROWS_PER_SUB, ROWS_PER_SUB), :],
    )

    @pl.when(s == NS - 1)
    def _():
        pltpu.sync_copy(
            acc_sh.at[pl.ds(NS * ROWS_PER_SUB, 16), :],
            out_hbm.at[c, pl.ds(NS * ROWS_PER_SUB, 16), :],
        )


@functools.lru_cache(maxsize=None)
def _scatter_call():
    return functools.partial(
        pl.kernel,
        out_type=jax.ShapeDtypeStruct((NC, N, D), jnp.float32),
        mesh=plsc.VectorSubcoreMesh(
            core_axis_name="c", subcore_axis_name="s", num_cores=NC, num_subcores=NS
        ),
        scratch_types=[
            pltpu.VMEM_SHARED((N, D), jnp.float32),
            pltpu.VMEM((NCHUNK, CHUNK), jnp.int32),
            pltpu.VMEM((RING, CHUNK, D), jnp.float32),
            pltpu.SemaphoreType.DMA((RING,)),
            pltpu.SemaphoreType.DMA((RING,)),
        ],
    )(_scatter_body)


BE = 2000


def _msg_body(xg_ref, ea_ref, w_ref, o_ref):
    x = xg_ref[...].astype(jnp.bfloat16)
    ea16 = ea_ref[...].astype(jnp.bfloat16)
    eab = jnp.concatenate(
        [jnp.broadcast_to(ea16[:, j : j + 1], (BE, D)) for j in range(D_EDGE)],
        axis=1,
    )
    ym = eab * jnp.tile(x, (1, D_EDGE))
    y = jnp.dot(ym, w_ref[...], preferred_element_type=jnp.float32) * (
        1.0 / np.sqrt(D * D_EDGE)
    )
    o_ref[...] = y * jax.nn.sigmoid(y)


def _msg_call(x_g, ea, w_t):
    return pl.pallas_call(
        _msg_body,
        grid=(E // BE,),
        in_specs=[
            pl.BlockSpec((BE, D), lambda i: (i, 0)),
            pl.BlockSpec((BE, D_EDGE), lambda i: (i, 0)),
            pl.BlockSpec((D_EDGE * D, D_HID), lambda i: (0, 0)),
        ],
        out_specs=pl.BlockSpec((BE, D_HID), lambda i: (i, 0)),
        out_shape=jax.ShapeDtypeStruct((E, D_HID), jnp.float32),
    )(x_g, ea, w_t)


BN = 400
IC = 16
NIC = D // IC


def _upd_body(sp_ref, x_ref, wlm_ref, w2f_ref, wlu_ref, o_ref):
    s = sp_ref[0] + sp_ref[1]
    agg = jnp.dot(s, wlm_ref[...], preferred_element_type=jnp.float32) * (
        1.0 / np.sqrt(D_HID)
    )
    x = x_ref[...]
    acc = jnp.zeros((BN, D_HID), jnp.float32)
    x16 = x.astype(jnp.bfloat16)
    agg_t = jnp.tile(agg.astype(jnp.bfloat16), (1, IC))
    for ic in range(NIC):
        xb = jnp.concatenate(
            [
                jnp.broadcast_to(x16[:, ic * IC + i : ic * IC + i + 1], (BN, D_HID))
                for i in range(IC)
            ],
            axis=1,
        )
        acc += jnp.dot(xb * agg_t, w2f_ref[ic], preferred_element_type=jnp.float32)
    y = acc * (1.0 / np.sqrt(D * D_HID))
    y = y * jax.nn.sigmoid(y)
    o_ref[...] = x + jnp.dot(y, wlu_ref[...], preferred_element_type=jnp.float32) * (
        1.0 / np.sqrt(D_HID)
    )


def _upd_call(s_parts, x, w_lm, w2f, w_lu):
    return pl.pallas_call(
        _upd_body,
        grid=(N // BN,),
        in_specs=[
            pl.BlockSpec((NC, BN, D), lambda i: (0, i, 0)),
            pl.BlockSpec((BN, D), lambda i: (i, 0)),
            pl.BlockSpec((D_HID, D_HID), lambda i: (0, 0)),
            pl.BlockSpec((NIC, IC * D_HID, D_HID), lambda i: (0, 0, 0)),
            pl.BlockSpec((D_HID, D), lambda i: (0, 0)),
        ],
        out_specs=pl.BlockSpec((BN, D), lambda i: (i, 0)),
        out_shape=jax.ShapeDtypeStruct((N, D), jnp.float32),
    )(s_parts, x, w_lm, w2f, w_lu)


@jax.jit
def kernel(node_features, edge_index, edge_attr_tensor, node_attr_scalar_raw,
           W_tp_msg, W_lin_msg, W_tp_upd, W_lin_upd):
    del node_attr_scalar_raw
    row = edge_index[0].reshape(NW, NCHUNK, CHUNK)
    col = edge_index[1].reshape(NW, NCHUNK, CHUNK)
    w_msg_t = (
        jnp.transpose(W_tp_msg, (1, 0, 2))
        .reshape(D_EDGE * D, D_HID)
        .astype(jnp.bfloat16)
    )
    w2f = W_tp_upd.reshape(NIC, IC * D_HID, D_HID).astype(jnp.bfloat16)
    zeros = jnp.zeros((ROWS_PER_SUB, D), jnp.float32)

    x_g = _gather_call()(node_features, col)
    msg = _msg_call(x_g, edge_attr_tensor, w_msg_t)
    s_parts = _scatter_call()(msg, row, zeros)
    return _upd_call(s_parts, node_features, W_lin_msg, w2f, W_lin_upd)
